# Initial kernel scaffold; baseline (speedup 1.0000x reference)
import jax
import jax.numpy as jnp
from jax.experimental import pallas as pl


def kernel(x, atomic_numbers, edge_distance, edge_index, src_table, tgt_table, W1, b1, g1, be1, W2, b2, g2, be2, W3, b3, to_m, wigner_inv):
    raise NotImplementedError("write your pallas kernel here")


# speedup vs baseline: 4.8695x; 4.8695x over previous
import functools

import jax
import jax.numpy as jnp
from jax.experimental import pallas as pl
from jax.experimental.pallas import tpu as pltpu

_LMAX = 2
_NUM_COEFF = (_LMAX + 1) ** 2
_M0 = _LMAX + 1
_C = 32
_RESCALE = 4.0
_L_IDX = (0, 2, 6)


def _ln(h, g, b):
    mu = jnp.mean(h, axis=-1, keepdims=True)
    var = jnp.mean((h - mu) ** 2, axis=-1, keepdims=True)
    return (h - mu) * jax.lax.rsqrt(var + 1e-5) * g + b


def _edge_body(dist_ref, src_ref, tgt_ref, wig_ref,
               w1_ref, w2_ref, w3_ref, vec_ref, b3_ref, out_ref):
    xe = jnp.concatenate([dist_ref[...], src_ref[...], tgt_ref[...]], axis=1)
    b1 = vec_ref[0, :]
    g1 = vec_ref[1, :]
    be1 = vec_ref[2, :]
    b2 = vec_ref[3, :]
    g2 = vec_ref[4, :]
    be2 = vec_ref[5, :]
    h = jnp.dot(xe, w1_ref[...], preferred_element_type=jnp.float32) + b1
    h = jax.nn.silu(_ln(h, g1, be1))
    h = jnp.dot(h, w2_ref[...], preferred_element_type=jnp.float32) + b2
    h = jax.nn.silu(_ln(h, g2, be2))
    h = jnp.dot(h, w3_ref[...], preferred_element_type=jnp.float32) + b3_ref[0, :]
    hk = [h[:, k * _C:(k + 1) * _C] for k in range(_M0)]
    for i in range(_NUM_COEFF):
        acc = None
        for k, l_idx in enumerate(_L_IDX):
            wk = wig_ref[:, 9 * i + l_idx:9 * i + l_idx + 1]
            term = wk * hk[k]
            acc = term if acc is None else acc + term
        out_ref[:, _C * i:_C * (i + 1)] = acc


def _edge_stage(dist, src, tgt, wigner_inv, w1t, w2t, w3t, vecs, b3s):
    e = dist.shape[0]
    blk = 1600
    grid = e // blk
    return pl.pallas_call(
        _edge_body,
        grid=(grid,),
        in_specs=[
            pl.BlockSpec((blk, 128), lambda i: (i, 0)),
            pl.BlockSpec((blk, 64), lambda i: (i, 0)),
            pl.BlockSpec((blk, 64), lambda i: (i, 0)),
            pl.BlockSpec((blk, _NUM_COEFF * _NUM_COEFF), lambda i: (i, 0)),
            pl.BlockSpec((256, 64), lambda i: (0, 0)),
            pl.BlockSpec((64, 64), lambda i: (0, 0)),
            pl.BlockSpec((64, _M0 * _C), lambda i: (0, 0)),
            pl.BlockSpec((8, 64), lambda i: (0, 0)),
            pl.BlockSpec((1, _M0 * _C), lambda i: (0, 0)),
        ],
        out_specs=pl.BlockSpec((blk, _NUM_COEFF * _C), lambda i: (i, 0)),
        out_shape=jax.ShapeDtypeStruct((e, _NUM_COEFF * _C), jnp.float32),
    )(dist, src, tgt, wigner_inv.reshape(e, _NUM_COEFF * _NUM_COEFF),
      w1t, w2t, w3t, vecs, b3s)


def kernel(x, atomic_numbers, edge_distance, edge_index, src_table, tgt_table,
           W1, b1, g1, be1, W2, b2, g2, be2, W3, b3, to_m, wigner_inv):
    n_nodes = x.shape[0]
    w1t = W1.T
    w2t = W2.T
    w3t = (W3 / _RESCALE).T
    b3s = (b3 / _RESCALE).reshape(1, _M0 * _C)
    vecs = jnp.stack([b1, g1, be1, b2, g2, be2, jnp.zeros_like(b1),
                      jnp.zeros_like(b1)], axis=0)
    elem_src = jnp.take(atomic_numbers, edge_index[0])
    elem_tgt = jnp.take(atomic_numbers, edge_index[1])
    src = jnp.take(src_table, elem_src, axis=0)
    tgt = jnp.take(tgt_table, elem_tgt, axis=0)

    emb = _edge_stage(edge_distance, src, tgt, wigner_inv,
                      w1t, w2t, w3t, vecs, b3s)

    out = jnp.zeros((n_nodes, _NUM_COEFF * _C), jnp.float32)
    out = out.at[edge_index[1]].add(emb)
    return out.reshape(n_nodes, _NUM_COEFF, _C)


# speedup vs baseline: 11.6359x; 2.3896x over previous
import functools

import jax
import jax.numpy as jnp
from jax import lax
from jax.experimental import pallas as pl
from jax.experimental.pallas import tpu as pltpu
from jax.experimental.pallas import tpu_sc as plsc

_NUM_COEFF = 9
_M0 = 3
_C = 32
_RESCALE = 4.0
_L_IDX = (0, 2, 6)
_HALF = (_NUM_COEFF * _C) // 2


def _sc_elem_gather(esrc, edst, an):
    e = esrc.shape[0]
    n = an.shape[0]
    per = e // 16
    mesh = plsc.VectorSubcoreMesh(core_axis_name="c", subcore_axis_name="s")

    @functools.partial(
        pl.kernel,
        out_type=[jax.ShapeDtypeStruct((e,), jnp.int32),
                  jax.ShapeDtypeStruct((e,), jnp.int32)],
        mesh=mesh,
        scratch_types=[pltpu.VMEM((per,), jnp.int32),
                       pltpu.VMEM((per,), jnp.int32),
                       pltpu.SemaphoreType.DMA],
    )
    def k(esrc_h, edst_h, an_h, es_h, et_h, idx_v, out_v, sem):
        cid = lax.axis_index("c")
        sid = lax.axis_index("s")
        base = sid * per

        @pl.when(cid == 0)
        def _():
            pltpu.sync_copy(esrc_h.at[pl.ds(base, per)], idx_v)

        @pl.when(cid == 1)
        def _():
            pltpu.sync_copy(edst_h.at[pl.ds(base, per)], idx_v)

        pltpu.async_copy(an_h.at[idx_v], out_v, sem).wait()

        @pl.when(cid == 0)
        def _():
            pltpu.sync_copy(out_v, es_h.at[pl.ds(base, per)])

        @pl.when(cid == 1)
        def _():
            pltpu.sync_copy(out_v, et_h.at[pl.ds(base, per)])

    return k(esrc, edst, an)


def _ln(h, g, b):
    mu = jnp.mean(h, axis=-1, keepdims=True)
    var = jnp.mean((h - mu) ** 2, axis=-1, keepdims=True)
    return (h - mu) * jax.lax.rsqrt(var + 1e-5) * g + b


def _edge_body(dist_ref, es_ref, et_ref, wig_ref,
               w1d_ref, s_ref, t_ref, w2_ref, w3_ref, vec_ref, b3_ref,
               lo_ref, hi_ref, rem_ref):
    blk = dist_ref.shape[0]
    b1 = vec_ref[0, :]
    g1 = vec_ref[1, :]
    be1 = vec_ref[2, :]
    b2 = vec_ref[3, :]
    g2 = vec_ref[4, :]
    be2 = vec_ref[5, :]
    ncls = s_ref.shape[0]
    cls_iota = jax.lax.broadcasted_iota(jnp.int32, (blk, ncls), 1)
    oh_s = (es_ref[...] == cls_iota).astype(jnp.float32)
    oh_t = (et_ref[...] == cls_iota).astype(jnp.float32)
    h = (jnp.dot(dist_ref[...], w1d_ref[...], preferred_element_type=jnp.float32)
         + jnp.dot(oh_s, s_ref[...], preferred_element_type=jnp.float32)
         + jnp.dot(oh_t, t_ref[...], preferred_element_type=jnp.float32)
         + b1)
    h = jax.nn.silu(_ln(h, g1, be1))
    h = jnp.dot(h, w2_ref[...], preferred_element_type=jnp.float32) + b2
    h = jax.nn.silu(_ln(h, g2, be2))
    h = jnp.dot(h, w3_ref[...], preferred_element_type=jnp.float32) + b3_ref[0, :]
    hk = [h[:, k * _C:(k + 1) * _C] for k in range(_M0)]
    coeffs = []
    for i in range(_NUM_COEFF):
        acc = None
        for k, l_idx in enumerate(_L_IDX):
            wk = wig_ref[:, 9 * i + l_idx:9 * i + l_idx + 1]
            term = wk * hk[k]
            acc = term if acc is None else acc + term
        coeffs.append(acc)
    lo_ref[...] = jnp.concatenate(coeffs[0:4], axis=1)
    hi_ref[...] = jnp.concatenate(coeffs[4:8], axis=1)
    rem_ref[...] = jnp.concatenate(
        [coeffs[8], jnp.zeros((blk, 96), jnp.float32)], axis=1)


def _edge_stage(dist, es, et, wig_flat, w1d, s96, t96, w2t, w3t, vecs, b3s):
    e = dist.shape[0]
    blk = 1600
    grid = e // blk
    return pl.pallas_call(
        _edge_body,
        grid=(grid,),
        in_specs=[
            pl.BlockSpec((blk, 128), lambda i: (i, 0)),
            pl.BlockSpec((blk, 1), lambda i: (i, 0)),
            pl.BlockSpec((blk, 1), lambda i: (i, 0)),
            pl.BlockSpec((blk, _NUM_COEFF * _NUM_COEFF), lambda i: (i, 0)),
            pl.BlockSpec((128, 64), lambda i: (0, 0)),
            pl.BlockSpec((96, 64), lambda i: (0, 0)),
            pl.BlockSpec((96, 64), lambda i: (0, 0)),
            pl.BlockSpec((64, 64), lambda i: (0, 0)),
            pl.BlockSpec((64, _M0 * _C), lambda i: (0, 0)),
            pl.BlockSpec((8, 64), lambda i: (0, 0)),
            pl.BlockSpec((1, _M0 * _C), lambda i: (0, 0)),
        ],
        out_specs=[pl.BlockSpec((blk, 128), lambda i: (i, 0)),
                   pl.BlockSpec((blk, 128), lambda i: (i, 0)),
                   pl.BlockSpec((blk, 128), lambda i: (i, 0))],
        out_shape=[jax.ShapeDtypeStruct((e, 128), jnp.float32),
                   jax.ShapeDtypeStruct((e, 128), jnp.float32),
                   jax.ShapeDtypeStruct((e, 128), jnp.float32)],
    )(dist, es, et, wig_flat, w1d, s96, t96, w2t, w3t, vecs, b3s)


def _sc_scatter_main(emb_a, emb_b, dst, n_nodes):
    e = dst.shape[0]
    per_edges = e // 16
    chunk = 200
    nchunk = per_edges // chunk
    zrows = 200
    nz_total = n_nodes // zrows
    nz_per = (nz_total + 15) // 16
    zeros = jnp.zeros((zrows, 128), jnp.float32)
    assert zrows == chunk
    mesh = plsc.VectorSubcoreMesh(core_axis_name="c", subcore_axis_name="s")

    @functools.partial(
        pl.kernel,
        out_type=[jax.ShapeDtypeStruct((n_nodes, 128), jnp.float32),
                  jax.ShapeDtypeStruct((n_nodes, 128), jnp.float32)],
        mesh=mesh,
        scratch_types=[pltpu.VMEM_SHARED((n_nodes, 128), jnp.float32),
                       pltpu.VMEM((chunk, 128), jnp.float32),
                       pltpu.VMEM((chunk,), jnp.int32)],
    )
    def k(a_h, b_h, dst_h, zero_h, out_a_h, out_b_h,
          acc, rows_v, idx_v):
        tmp_v = rows_v
        cid = lax.axis_index("c")
        sid = lax.axis_index("s")
        pltpu.sync_copy(zero_h, tmp_v)
        for z in range(nz_per):
            cidx = sid + 16 * z

            @pl.when(cidx < nz_total)
            def _():
                pltpu.sync_copy(tmp_v, acc.at[pl.ds(cidx * zrows, zrows)])

        plsc.subcore_barrier()
        ebase = sid * per_edges
        for ci in range(nchunk):
            b = ebase + ci * chunk
            pltpu.sync_copy(dst_h.at[pl.ds(b, chunk)], idx_v)

            @pl.when(cid == 0)
            def _():
                pltpu.sync_copy(a_h.at[pl.ds(b, chunk)], rows_v)

            @pl.when(cid == 1)
            def _():
                pltpu.sync_copy(b_h.at[pl.ds(b, chunk)], rows_v)

            pltpu.sync_copy(rows_v, acc.at[idx_v], add=True)
        plsc.subcore_barrier()
        for z in range(nz_per):
            cidx = sid + 16 * z

            @pl.when(cidx < nz_total)
            def _():
                r0 = cidx * zrows
                pltpu.sync_copy(acc.at[pl.ds(r0, zrows)], tmp_v)

                @pl.when(cid == 0)
                def _():
                    pltpu.sync_copy(tmp_v, out_a_h.at[pl.ds(r0, zrows)])

                @pl.when(cid == 1)
                def _():
                    pltpu.sync_copy(tmp_v, out_b_h.at[pl.ds(r0, zrows)])

    return k(emb_a, emb_b, dst, zeros)


def _sc_scatter_rem(emb_c, dst, n_nodes):
    e = dst.shape[0]
    per_edges = e // 32
    chunk = 200
    nchunk = per_edges // chunk
    zrows = 200
    nz_total = n_nodes // zrows
    nz_per = (nz_total + 15) // 16
    zeros = jnp.zeros((zrows, 128), jnp.float32)
    assert zrows == chunk
    mesh = plsc.VectorSubcoreMesh(core_axis_name="c", subcore_axis_name="s")

    @functools.partial(
        pl.kernel,
        out_type=[jax.ShapeDtypeStruct((n_nodes, 128), jnp.float32),
                  jax.ShapeDtypeStruct((n_nodes, 128), jnp.float32)],
        mesh=mesh,
        scratch_types=[pltpu.VMEM_SHARED((n_nodes, 128), jnp.float32),
                       pltpu.VMEM((chunk, 128), jnp.float32),
                       pltpu.VMEM((chunk,), jnp.int32)],
    )
    def k(c_h, dst_h, zero_h, out0_h, out1_h, acc, rows_v, idx_v):
        tmp_v = rows_v
        cid = lax.axis_index("c")
        sid = lax.axis_index("s")
        pltpu.sync_copy(zero_h, tmp_v)
        for z in range(nz_per):
            cidx = sid + 16 * z

            @pl.when(cidx < nz_total)
            def _():
                pltpu.sync_copy(tmp_v, acc.at[pl.ds(cidx * zrows, zrows)])

        plsc.subcore_barrier()
        ebase = (cid * 16 + sid) * per_edges
        for ci in range(nchunk):
            b = ebase + ci * chunk
            pltpu.sync_copy(dst_h.at[pl.ds(b, chunk)], idx_v)
            pltpu.sync_copy(c_h.at[pl.ds(b, chunk)], rows_v)
            pltpu.sync_copy(rows_v, acc.at[idx_v], add=True)
        plsc.subcore_barrier()
        for z in range(nz_per):
            cidx = sid + 16 * z

            @pl.when(cidx < nz_total)
            def _():
                r0 = cidx * zrows
                pltpu.sync_copy(acc.at[pl.ds(r0, zrows)], tmp_v)

                @pl.when(cid == 0)
                def _():
                    pltpu.sync_copy(tmp_v, out0_h.at[pl.ds(r0, zrows)])

                @pl.when(cid == 1)
                def _():
                    pltpu.sync_copy(tmp_v, out1_h.at[pl.ds(r0, zrows)])

    return k(emb_c, dst, zeros)


def kernel(x, atomic_numbers, edge_distance, edge_index, src_table, tgt_table,
           W1, b1, g1, be1, W2, b2, g2, be2, W3, b3, to_m, wigner_inv):
    n_nodes = x.shape[0]
    e = edge_distance.shape[0]
    w1d = W1[:, :128].T
    s96 = jnp.zeros((96, 64), jnp.float32).at[:src_table.shape[0]].set(
        src_table @ W1[:, 128:192].T)
    t96 = jnp.zeros((96, 64), jnp.float32).at[:tgt_table.shape[0]].set(
        tgt_table @ W1[:, 192:256].T)
    w2t = W2.T
    w3t = (W3 / _RESCALE).T
    b3s = (b3 / _RESCALE).reshape(1, _M0 * _C)
    vecs = jnp.stack([b1, g1, be1, b2, g2, be2, jnp.zeros_like(b1),
                      jnp.zeros_like(b1)], axis=0)
    esrc = edge_index[0].astype(jnp.int32)
    edst = edge_index[1].astype(jnp.int32)
    an32 = atomic_numbers.astype(jnp.int32)

    es, et = _sc_elem_gather(esrc, edst, an32)
    emb_a, emb_b, emb_c = _edge_stage(
        edge_distance, es.reshape(e, 1), et.reshape(e, 1),
        wigner_inv.reshape(e, _NUM_COEFF * _NUM_COEFF),
        w1d, s96, t96, w2t, w3t, vecs, b3s)
    out_a, out_b = _sc_scatter_main(emb_a, emb_b, edst, n_nodes)
    out_c0, out_c1 = _sc_scatter_rem(emb_c, edst, n_nodes)
    out_c = (out_c0 + out_c1)[:, :_C]
    out = jnp.concatenate([out_a, out_b, out_c], axis=1)
    return out.reshape(n_nodes, _NUM_COEFF, _C)


# speedup vs baseline: 20.7758x; 1.7855x over previous
import functools

import numpy as np
import jax
import jax.numpy as jnp
from jax import lax
from jax.experimental import pallas as pl
from jax.experimental.pallas import tpu as pltpu
from jax.experimental.pallas import tpu_sc as plsc

_NUM_COEFF = 9
_M0 = 3
_C = 32
_RESCALE = 4.0
_L_IDX = (0, 2, 6)
_HALF = (_NUM_COEFF * _C) // 2


def _sc_elem_gather(esrc, edst, an):
    e = esrc.shape[0]
    n = an.shape[0]
    per = e // 16
    mesh = plsc.VectorSubcoreMesh(core_axis_name="c", subcore_axis_name="s")

    @functools.partial(
        pl.kernel,
        out_type=[jax.ShapeDtypeStruct((e,), jnp.int32),
                  jax.ShapeDtypeStruct((e,), jnp.int32)],
        mesh=mesh,
        scratch_types=[pltpu.VMEM((per,), jnp.int32),
                       pltpu.VMEM((per,), jnp.int32),
                       pltpu.SemaphoreType.DMA],
    )
    def k(esrc_h, edst_h, an_h, es_h, et_h, idx_v, out_v, sem):
        cid = lax.axis_index("c")
        sid = lax.axis_index("s")
        base = sid * per

        @pl.when(cid == 0)
        def _():
            pltpu.sync_copy(esrc_h.at[pl.ds(base, per)], idx_v)

        @pl.when(cid == 1)
        def _():
            pltpu.sync_copy(edst_h.at[pl.ds(base, per)], idx_v)

        pltpu.async_copy(an_h.at[idx_v], out_v, sem).wait()

        @pl.when(cid == 0)
        def _():
            pltpu.sync_copy(out_v, es_h.at[pl.ds(base, per)])

        @pl.when(cid == 1)
        def _():
            pltpu.sync_copy(out_v, et_h.at[pl.ds(base, per)])

    return k(esrc, edst, an)


def _ln(h, g, b):
    mu = jnp.mean(h, axis=-1, keepdims=True)
    var = jnp.mean((h - mu) ** 2, axis=-1, keepdims=True)
    return (h - mu) * jax.lax.rsqrt(var + 1e-5) * g + b


def _edge_body(dist_ref, es_ref, et_ref, wig_ref,
               w1d_ref, s_ref, t_ref, w2_ref, w3rep_ref, e3_ref,
               vec_ref, b3rep_ref,
               lo_ref, hi_ref, rem_ref):
    blk = dist_ref.shape[0]
    b1 = vec_ref[0, :]
    g1 = vec_ref[1, :]
    be1 = vec_ref[2, :]
    b2 = vec_ref[3, :]
    g2 = vec_ref[4, :]
    be2 = vec_ref[5, :]
    ncls = s_ref.shape[0]
    cls_iota = jax.lax.broadcasted_iota(jnp.int32, (blk, ncls), 1)
    oh_s = (es_ref[...] == cls_iota).astype(jnp.float32)
    oh_t = (et_ref[...] == cls_iota).astype(jnp.float32)
    h = (jnp.dot(dist_ref[...], w1d_ref[...], preferred_element_type=jnp.float32)
         + jnp.dot(oh_s, s_ref[...], preferred_element_type=jnp.float32)
         + jnp.dot(oh_t, t_ref[...], preferred_element_type=jnp.float32)
         + b1)
    h = jax.nn.silu(_ln(h, g1, be1))
    h = jnp.dot(h, w2_ref[...], preferred_element_type=jnp.float32) + b2
    h = jax.nn.silu(_ln(h, g2, be2))
    wig = wig_ref[...]
    emb = None
    for k in range(_M0):
        hk = jnp.dot(h, w3rep_ref[k], preferred_element_type=jnp.float32) \
            + b3rep_ref[k, :]
        wk = jnp.dot(wig, e3_ref[k], preferred_element_type=jnp.float32)
        term = wk * hk
        emb = term if emb is None else emb + term
    lo_ref[...] = emb[:, 0:128]
    hi_ref[...] = emb[:, 128:256]
    rem_ref[...] = jnp.concatenate(
        [emb[:, 256:288], jnp.zeros((blk, 96), jnp.float32)], axis=1)


def _edge_stage(dist, es, et, wig_flat, w1d, s96, t96, w2t, w3rep, e3, vecs,
                b3rep):
    e = dist.shape[0]
    blk = 1600
    grid = e // blk
    nc2 = _NUM_COEFF * _NUM_COEFF
    return pl.pallas_call(
        _edge_body,
        grid=(grid,),
        in_specs=[
            pl.BlockSpec((blk, 128), lambda i: (i, 0)),
            pl.BlockSpec((blk, 1), lambda i: (i, 0)),
            pl.BlockSpec((blk, 1), lambda i: (i, 0)),
            pl.BlockSpec((blk, nc2), lambda i: (i, 0)),
            pl.BlockSpec((128, 64), lambda i: (0, 0)),
            pl.BlockSpec((96, 64), lambda i: (0, 0)),
            pl.BlockSpec((96, 64), lambda i: (0, 0)),
            pl.BlockSpec((64, 64), lambda i: (0, 0)),
            pl.BlockSpec((_M0, 64, 288), lambda i: (0, 0, 0)),
            pl.BlockSpec((_M0, nc2, 288), lambda i: (0, 0, 0)),
            pl.BlockSpec((8, 64), lambda i: (0, 0)),
            pl.BlockSpec((_M0, 288), lambda i: (0, 0)),
        ],
        out_specs=[pl.BlockSpec((blk, 128), lambda i: (i, 0)),
                   pl.BlockSpec((blk, 128), lambda i: (i, 0)),
                   pl.BlockSpec((blk, 128), lambda i: (i, 0))],
        out_shape=[jax.ShapeDtypeStruct((e, 128), jnp.float32),
                   jax.ShapeDtypeStruct((e, 128), jnp.float32),
                   jax.ShapeDtypeStruct((e, 128), jnp.float32)],
    )(dist, es, et, wig_flat, w1d, s96, t96, w2t, w3rep, e3, vecs, b3rep)


def _sc_scatter_main(emb_a, emb_b, dst, n_nodes):
    e = dst.shape[0]
    per_edges = e // 16
    chunk = 200
    nchunk = per_edges // chunk
    zrows = 200
    nz_total = n_nodes // zrows
    nz_per = (nz_total + 15) // 16
    zeros = jnp.zeros((zrows, 128), jnp.float32)
    assert zrows == chunk
    mesh = plsc.VectorSubcoreMesh(core_axis_name="c", subcore_axis_name="s")

    @functools.partial(
        pl.kernel,
        out_type=[jax.ShapeDtypeStruct((n_nodes, 128), jnp.float32),
                  jax.ShapeDtypeStruct((n_nodes, 128), jnp.float32)],
        mesh=mesh,
        scratch_types=[pltpu.VMEM_SHARED((n_nodes, 128), jnp.float32),
                       pltpu.VMEM((chunk, 128), jnp.float32),
                       pltpu.VMEM((chunk,), jnp.int32)],
    )
    def k(a_h, b_h, dst_h, zero_h, out_a_h, out_b_h,
          acc, rows_v, idx_v):
        tmp_v = rows_v
        cid = lax.axis_index("c")
        sid = lax.axis_index("s")
        pltpu.sync_copy(zero_h, tmp_v)
        for z in range(nz_per):
            cidx = sid + 16 * z

            @pl.when(cidx < nz_total)
            def _():
                pltpu.sync_copy(tmp_v, acc.at[pl.ds(cidx * zrows, zrows)])

        plsc.subcore_barrier()
        ebase = sid * per_edges
        for ci in range(nchunk):
            b = ebase + ci * chunk
            pltpu.sync_copy(dst_h.at[pl.ds(b, chunk)], idx_v)

            @pl.when(cid == 0)
            def _():
                pltpu.sync_copy(a_h.at[pl.ds(b, chunk)], rows_v)

            @pl.when(cid == 1)
            def _():
                pltpu.sync_copy(b_h.at[pl.ds(b, chunk)], rows_v)

            pltpu.sync_copy(rows_v, acc.at[idx_v], add=True)
        plsc.subcore_barrier()
        for z in range(nz_per):
            cidx = sid + 16 * z

            @pl.when(cidx < nz_total)
            def _():
                r0 = cidx * zrows
                pltpu.sync_copy(acc.at[pl.ds(r0, zrows)], tmp_v)

                @pl.when(cid == 0)
                def _():
                    pltpu.sync_copy(tmp_v, out_a_h.at[pl.ds(r0, zrows)])

                @pl.when(cid == 1)
                def _():
                    pltpu.sync_copy(tmp_v, out_b_h.at[pl.ds(r0, zrows)])

    return k(emb_a, emb_b, dst, zeros)


def _sc_scatter_rem(emb_c, dst, n_nodes):
    e = dst.shape[0]
    per_edges = e // 32
    chunk = 200
    nchunk = per_edges // chunk
    zrows = 200
    nz_total = n_nodes // zrows
    nz_per = (nz_total + 15) // 16
    zeros = jnp.zeros((zrows, 128), jnp.float32)
    assert zrows == chunk
    mesh = plsc.VectorSubcoreMesh(core_axis_name="c", subcore_axis_name="s")

    @functools.partial(
        pl.kernel,
        out_type=[jax.ShapeDtypeStruct((n_nodes, 128), jnp.float32),
                  jax.ShapeDtypeStruct((n_nodes, 128), jnp.float32)],
        mesh=mesh,
        scratch_types=[pltpu.VMEM_SHARED((n_nodes, 128), jnp.float32),
                       pltpu.VMEM((chunk, 128), jnp.float32),
                       pltpu.VMEM((chunk,), jnp.int32)],
    )
    def k(c_h, dst_h, zero_h, out0_h, out1_h, acc, rows_v, idx_v):
        tmp_v = rows_v
        cid = lax.axis_index("c")
        sid = lax.axis_index("s")
        pltpu.sync_copy(zero_h, tmp_v)
        for z in range(nz_per):
            cidx = sid + 16 * z

            @pl.when(cidx < nz_total)
            def _():
                pltpu.sync_copy(tmp_v, acc.at[pl.ds(cidx * zrows, zrows)])

        plsc.subcore_barrier()
        ebase = (cid * 16 + sid) * per_edges
        for ci in range(nchunk):
            b = ebase + ci * chunk
            pltpu.sync_copy(dst_h.at[pl.ds(b, chunk)], idx_v)
            pltpu.sync_copy(c_h.at[pl.ds(b, chunk)], rows_v)
            pltpu.sync_copy(rows_v, acc.at[idx_v], add=True)
        plsc.subcore_barrier()
        for z in range(nz_per):
            cidx = sid + 16 * z

            @pl.when(cidx < nz_total)
            def _():
                r0 = cidx * zrows
                pltpu.sync_copy(acc.at[pl.ds(r0, zrows)], tmp_v)

                @pl.when(cid == 0)
                def _():
                    pltpu.sync_copy(tmp_v, out0_h.at[pl.ds(r0, zrows)])

                @pl.when(cid == 1)
                def _():
                    pltpu.sync_copy(tmp_v, out1_h.at[pl.ds(r0, zrows)])

    return k(emb_c, dst, zeros)


def kernel(x, atomic_numbers, edge_distance, edge_index, src_table, tgt_table,
           W1, b1, g1, be1, W2, b2, g2, be2, W3, b3, to_m, wigner_inv):
    n_nodes = x.shape[0]
    e = edge_distance.shape[0]
    w1d = W1[:, :128].T
    s96 = jnp.zeros((96, 64), jnp.float32).at[:src_table.shape[0]].set(
        src_table @ W1[:, 128:192].T)
    t96 = jnp.zeros((96, 64), jnp.float32).at[:tgt_table.shape[0]].set(
        tgt_table @ W1[:, 192:256].T)
    w2t = W2.T
    w3t = (W3 / _RESCALE).T
    b3s = (b3 / _RESCALE).reshape(_M0, _C)
    w3rep = jnp.stack([jnp.tile(w3t[:, k * _C:(k + 1) * _C], (1, _NUM_COEFF))
                       for k in range(_M0)], axis=0)
    b3rep = jnp.stack([jnp.tile(b3s[k], (_NUM_COEFF,))
                       for k in range(_M0)], axis=0)
    e3_np = np.zeros((_M0, _NUM_COEFF * _NUM_COEFF, _NUM_COEFF * _C),
                      dtype=np.float32)
    for k, l_idx in enumerate(_L_IDX):
        for i in range(_NUM_COEFF):
            e3_np[k, 9 * i + l_idx, _C * i:_C * (i + 1)] = 1.0
    e3 = jnp.asarray(e3_np)
    vecs = jnp.stack([b1, g1, be1, b2, g2, be2, jnp.zeros_like(b1),
                      jnp.zeros_like(b1)], axis=0)
    esrc = edge_index[0].astype(jnp.int32)
    edst = edge_index[1].astype(jnp.int32)
    an32 = atomic_numbers.astype(jnp.int32)

    es, et = _sc_elem_gather(esrc, edst, an32)
    emb_a, emb_b, emb_c = _edge_stage(
        edge_distance, es.reshape(e, 1), et.reshape(e, 1),
        wigner_inv.reshape(e, _NUM_COEFF * _NUM_COEFF),
        w1d, s96, t96, w2t, w3rep, e3, vecs, b3rep)
    out_a, out_b = _sc_scatter_main(emb_a, emb_b, edst, n_nodes)
    out_c0, out_c1 = _sc_scatter_rem(emb_c, edst, n_nodes)
    out_c = (out_c0 + out_c1)[:, :_C]
    out = jnp.concatenate([out_a, out_b, out_c], axis=1)
    return out.reshape(n_nodes, _NUM_COEFF, _C)


# speedup vs baseline: 21.2868x; 1.0246x over previous
import functools

import numpy as np
import jax
import jax.numpy as jnp
from jax import lax
from jax.experimental import pallas as pl
from jax.experimental.pallas import tpu as pltpu
from jax.experimental.pallas import tpu_sc as plsc

_NUM_COEFF = 9
_M0 = 3
_C = 32
_RESCALE = 4.0
_L_IDX = (0, 2, 6)
_HALF = (_NUM_COEFF * _C) // 2


def _sc_elem_gather(esrc, edst, an):
    e = esrc.shape[0]
    n = an.shape[0]
    per = e // 16
    mesh = plsc.VectorSubcoreMesh(core_axis_name="c", subcore_axis_name="s")

    @functools.partial(
        pl.kernel,
        out_type=[jax.ShapeDtypeStruct((e,), jnp.int32),
                  jax.ShapeDtypeStruct((e,), jnp.int32)],
        mesh=mesh,
        scratch_types=[pltpu.VMEM((per,), jnp.int32),
                       pltpu.VMEM((per,), jnp.int32),
                       pltpu.SemaphoreType.DMA],
    )
    def k(esrc_h, edst_h, an_h, es_h, et_h, idx_v, out_v, sem):
        cid = lax.axis_index("c")
        sid = lax.axis_index("s")
        base = sid * per

        @pl.when(cid == 0)
        def _():
            pltpu.sync_copy(esrc_h.at[pl.ds(base, per)], idx_v)

        @pl.when(cid == 1)
        def _():
            pltpu.sync_copy(edst_h.at[pl.ds(base, per)], idx_v)

        pltpu.async_copy(an_h.at[idx_v], out_v, sem).wait()

        @pl.when(cid == 0)
        def _():
            pltpu.sync_copy(out_v, es_h.at[pl.ds(base, per)])

        @pl.when(cid == 1)
        def _():
            pltpu.sync_copy(out_v, et_h.at[pl.ds(base, per)])

    return k(esrc, edst, an)


def _ln(h, g, b):
    mu = jnp.mean(h, axis=-1, keepdims=True)
    var = jnp.mean((h - mu) ** 2, axis=-1, keepdims=True)
    return (h - mu) * jax.lax.rsqrt(var + 1e-5) * g + b


def _edge_body(dist_ref, es_ref, et_ref, ed_ref, wig_ref,
               w1d_ref, s_ref, t_ref, w2_ref, w3rep_ref, e3_ref,
               vec_ref, b3rep_ref, p4_ref, r4_ref,
               lo_ref, hi_ref, rem_ref):
    blk = dist_ref.shape[0]
    b1 = vec_ref[0, :]
    g1 = vec_ref[1, :]
    be1 = vec_ref[2, :]
    b2 = vec_ref[3, :]
    g2 = vec_ref[4, :]
    be2 = vec_ref[5, :]
    ncls = s_ref.shape[0]
    cls_iota = jax.lax.broadcasted_iota(jnp.int32, (blk, ncls), 1)
    oh_s = (es_ref[...] == cls_iota).astype(jnp.float32)
    oh_t = (et_ref[...] == cls_iota).astype(jnp.float32)
    h = (jnp.dot(dist_ref[...], w1d_ref[...], preferred_element_type=jnp.float32)
         + jnp.dot(oh_s, s_ref[...], preferred_element_type=jnp.float32)
         + jnp.dot(oh_t, t_ref[...], preferred_element_type=jnp.float32)
         + b1)
    h = jax.nn.silu(_ln(h, g1, be1))
    h = jnp.dot(h, w2_ref[...], preferred_element_type=jnp.float32) + b2
    h = jax.nn.silu(_ln(h, g2, be2))
    wig = wig_ref[...]
    emb = None
    for k in range(_M0):
        hk = jnp.dot(h, w3rep_ref[k], preferred_element_type=jnp.float32) \
            + b3rep_ref[k, :]
        wk = jnp.dot(wig, e3_ref[k], preferred_element_type=jnp.float32)
        term = wk * hk
        emb = term if emb is None else emb + term
    lo_ref[...] = emb[:, 0:128]
    hi_ref[...] = emb[:, 128:256]
    md = jnp.bitwise_and(ed_ref[...], 3)
    oh4 = (md == jax.lax.broadcasted_iota(jnp.int32, (blk, 4), 1))
    w4 = jnp.dot(oh4.astype(jnp.float32), p4_ref[...],
                 preferred_element_type=jnp.float32)
    h4 = jnp.dot(emb[:, 256:288], r4_ref[...],
                 preferred_element_type=jnp.float32)
    rem_ref[...] = w4 * h4


def _edge_stage(dist, es, et, ed, wig_flat, w1d, s96, t96, w2t, w3rep, e3,
                vecs, b3rep, p4, r4):
    e = dist.shape[0]
    blk = 1600
    grid = e // blk
    nc2 = _NUM_COEFF * _NUM_COEFF
    return pl.pallas_call(
        _edge_body,
        grid=(grid,),
        in_specs=[
            pl.BlockSpec((blk, 128), lambda i: (i, 0)),
            pl.BlockSpec((blk, 1), lambda i: (i, 0)),
            pl.BlockSpec((blk, 1), lambda i: (i, 0)),
            pl.BlockSpec((blk, 1), lambda i: (i, 0)),
            pl.BlockSpec((blk, nc2), lambda i: (i, 0)),
            pl.BlockSpec((128, 64), lambda i: (0, 0)),
            pl.BlockSpec((96, 64), lambda i: (0, 0)),
            pl.BlockSpec((96, 64), lambda i: (0, 0)),
            pl.BlockSpec((64, 64), lambda i: (0, 0)),
            pl.BlockSpec((_M0, 64, 288), lambda i: (0, 0, 0)),
            pl.BlockSpec((_M0, nc2, 288), lambda i: (0, 0, 0)),
            pl.BlockSpec((8, 64), lambda i: (0, 0)),
            pl.BlockSpec((_M0, 288), lambda i: (0, 0)),
            pl.BlockSpec((4, 128), lambda i: (0, 0)),
            pl.BlockSpec((32, 128), lambda i: (0, 0)),
        ],
        out_specs=[pl.BlockSpec((blk, 128), lambda i: (i, 0)),
                   pl.BlockSpec((blk, 128), lambda i: (i, 0)),
                   pl.BlockSpec((blk, 128), lambda i: (i, 0))],
        out_shape=[jax.ShapeDtypeStruct((e, 128), jnp.float32),
                   jax.ShapeDtypeStruct((e, 128), jnp.float32),
                   jax.ShapeDtypeStruct((e, 128), jnp.float32)],
    )(dist, es, et, ed, wig_flat, w1d, s96, t96, w2t, w3rep, e3, vecs, b3rep,
      p4, r4)


_CK = 128


def _sc_scatter_main(emb_a, emb_b, dst_pad, n_nodes):
    e = emb_a.shape[0]
    nchunks = e // _CK
    base_per = nchunks // 16
    extra = nchunks - base_per * 16
    per_max = base_per + (1 if extra else 0)
    assert dst_pad.shape[0] >= base_per * 15 + min(15, extra) + per_max
    nfull = n_nodes // _CK
    tail = n_nodes - nfull * _CK
    zeros = jnp.zeros((_CK, 128), jnp.float32)
    mesh = plsc.VectorSubcoreMesh(core_axis_name="c", subcore_axis_name="s")

    @functools.partial(
        pl.kernel,
        out_type=[jax.ShapeDtypeStruct((n_nodes, 128), jnp.float32),
                  jax.ShapeDtypeStruct((n_nodes, 128), jnp.float32)],
        mesh=mesh,
        scratch_types=[pltpu.VMEM_SHARED((n_nodes, 128), jnp.float32),
                       pltpu.VMEM((per_max, 1, _CK), jnp.int32),
                       pltpu.VMEM((2, _CK, 128), jnp.float32),
                       pltpu.VMEM((max(tail, 8), 128), jnp.float32),
                       pltpu.SemaphoreType.DMA,
                       pltpu.SemaphoreType.DMA],
    )
    def k(a_h, b_h, dst_h, zero_h, out_a_h, out_b_h,
          acc, idx_v, rows_v, tail_v, sem0, sem1):
        cid = lax.axis_index("c")
        sid = lax.axis_index("s")
        sems = (sem0, sem1)
        start = base_per * sid + jnp.minimum(sid, extra)
        cnt = base_per + (sid < extra).astype(jnp.int32)
        pltpu.sync_copy(zero_h, rows_v.at[0])
        for z in range((nfull + 15) // 16):
            cidx = sid + 16 * z

            @pl.when(cidx < nfull)
            def _():
                pltpu.sync_copy(rows_v.at[0], acc.at[pl.ds(cidx * _CK, _CK)])

        if tail:
            @pl.when(sid == 0)
            def _():
                pltpu.sync_copy(zero_h.at[pl.ds(0, tail)], tail_v)
                pltpu.sync_copy(tail_v, acc.at[pl.ds(nfull * _CK, tail)])

        pltpu.sync_copy(dst_h.at[pl.ds(start, per_max)], idx_v)
        plsc.subcore_barrier()

        def start_load(j):
            c = start + j
            buf = j % 2

            @pl.when(j < cnt)
            def _():
                @pl.when(cid == 0)
                def _():
                    pltpu.async_copy(a_h.at[pl.ds(c * _CK, _CK)],
                                     rows_v.at[buf], sems[buf])

                @pl.when(cid == 1)
                def _():
                    pltpu.async_copy(b_h.at[pl.ds(c * _CK, _CK)],
                                     rows_v.at[buf], sems[buf])

        start_load(0)
        for j in range(per_max):
            buf = j % 2

            @pl.when(j < cnt)
            def _():
                pltpu.make_async_copy(a_h.at[pl.ds(0, _CK)],
                                      rows_v.at[buf], sems[buf]).wait()

            if j + 1 < per_max:
                start_load(j + 1)

            @pl.when(j < cnt)
            def _():
                pltpu.sync_copy(rows_v.at[buf], acc.at[idx_v.at[j, 0]], add=True)

        plsc.subcore_barrier()
        for z in range((nfull + 15) // 16):
            cidx = sid + 16 * z

            @pl.when(cidx < nfull)
            def _():
                r0 = cidx * _CK
                pltpu.sync_copy(acc.at[pl.ds(r0, _CK)], rows_v.at[0])

                @pl.when(cid == 0)
                def _():
                    pltpu.sync_copy(rows_v.at[0], out_a_h.at[pl.ds(r0, _CK)])

                @pl.when(cid == 1)
                def _():
                    pltpu.sync_copy(rows_v.at[0], out_b_h.at[pl.ds(r0, _CK)])

        if tail:
            @pl.when(sid == 1)
            def _():
                r0 = nfull * _CK
                pltpu.sync_copy(acc.at[pl.ds(r0, tail)], tail_v)

                @pl.when(cid == 0)
                def _():
                    pltpu.sync_copy(tail_v, out_a_h.at[pl.ds(r0, tail)])

                @pl.when(cid == 1)
                def _():
                    pltpu.sync_copy(tail_v, out_b_h.at[pl.ds(r0, tail)])

    return k(emb_a, emb_b, dst_pad, zeros)


def _sc_scatter_rem(emb_c4, dstq_pad, nq):
    e = emb_c4.shape[0]
    nchunks = e // _CK
    half = nchunks // 2
    base_per = half // 16
    extra = half - base_per * 16
    per_max = base_per + (1 if extra else 0)
    nfull = nq // _CK
    assert nq % _CK == 0
    zeros = jnp.zeros((_CK, 128), jnp.float32)
    mesh = plsc.VectorSubcoreMesh(core_axis_name="c", subcore_axis_name="s")

    @functools.partial(
        pl.kernel,
        out_type=[jax.ShapeDtypeStruct((nq, 128), jnp.float32),
                  jax.ShapeDtypeStruct((nq, 128), jnp.float32)],
        mesh=mesh,
        scratch_types=[pltpu.VMEM_SHARED((nq, 128), jnp.float32),
                       pltpu.VMEM((per_max, 1, _CK), jnp.int32),
                       pltpu.VMEM((2, _CK, 128), jnp.float32),
                       pltpu.SemaphoreType.DMA,
                       pltpu.SemaphoreType.DMA],
    )
    def k(c_h, dstq_h, zero_h, out0_h, out1_h, acc, idx_v, rows_v, sem0, sem1):
        cid = lax.axis_index("c")
        sid = lax.axis_index("s")
        sems = (sem0, sem1)
        start = cid * half + base_per * sid + jnp.minimum(sid, extra)
        cnt = base_per + (sid < extra).astype(jnp.int32)
        pltpu.sync_copy(zero_h, rows_v.at[0])
        for z in range((nfull + 15) // 16):
            cidx = sid + 16 * z

            @pl.when(cidx < nfull)
            def _():
                pltpu.sync_copy(rows_v.at[0], acc.at[pl.ds(cidx * _CK, _CK)])

        pltpu.sync_copy(dstq_h.at[pl.ds(start, per_max)], idx_v)
        plsc.subcore_barrier()

        def start_load(j):
            c = start + j
            buf = j % 2

            @pl.when(j < cnt)
            def _():
                pltpu.async_copy(c_h.at[pl.ds(c * _CK, _CK)],
                                 rows_v.at[buf], sems[buf])

        start_load(0)
        for j in range(per_max):
            buf = j % 2

            @pl.when(j < cnt)
            def _():
                pltpu.make_async_copy(c_h.at[pl.ds(0, _CK)],
                                      rows_v.at[buf], sems[buf]).wait()

            if j + 1 < per_max:
                start_load(j + 1)

            @pl.when(j < cnt)
            def _():
                pltpu.sync_copy(rows_v.at[buf], acc.at[idx_v.at[j, 0]], add=True)

        plsc.subcore_barrier()
        for z in range((nfull + 15) // 16):
            cidx = sid + 16 * z

            @pl.when(cidx < nfull)
            def _():
                r0 = cidx * _CK
                pltpu.sync_copy(acc.at[pl.ds(r0, _CK)], rows_v.at[0])

                @pl.when(cid == 0)
                def _():
                    pltpu.sync_copy(rows_v.at[0], out0_h.at[pl.ds(r0, _CK)])

                @pl.when(cid == 1)
                def _():
                    pltpu.sync_copy(rows_v.at[0], out1_h.at[pl.ds(r0, _CK)])

    return k(emb_c4, dstq_pad, zeros)


def kernel(x, atomic_numbers, edge_distance, edge_index, src_table, tgt_table,
           W1, b1, g1, be1, W2, b2, g2, be2, W3, b3, to_m, wigner_inv):
    n_nodes = x.shape[0]
    e = edge_distance.shape[0]
    w1d = W1[:, :128].T
    s96 = jnp.zeros((96, 64), jnp.float32).at[:src_table.shape[0]].set(
        src_table @ W1[:, 128:192].T)
    t96 = jnp.zeros((96, 64), jnp.float32).at[:tgt_table.shape[0]].set(
        tgt_table @ W1[:, 192:256].T)
    w2t = W2.T
    w3t = (W3 / _RESCALE).T
    b3s = (b3 / _RESCALE).reshape(_M0, _C)
    w3rep = jnp.stack([jnp.tile(w3t[:, k * _C:(k + 1) * _C], (1, _NUM_COEFF))
                       for k in range(_M0)], axis=0)
    b3rep = jnp.stack([jnp.tile(b3s[k], (_NUM_COEFF,))
                       for k in range(_M0)], axis=0)
    e3_np = np.zeros((_M0, _NUM_COEFF * _NUM_COEFF, _NUM_COEFF * _C),
                      dtype=np.float32)
    for k, l_idx in enumerate(_L_IDX):
        for i in range(_NUM_COEFF):
            e3_np[k, 9 * i + l_idx, _C * i:_C * (i + 1)] = 1.0
    e3 = jnp.asarray(e3_np)
    vecs = jnp.stack([b1, g1, be1, b2, g2, be2, jnp.zeros_like(b1),
                      jnp.zeros_like(b1)], axis=0)
    p4_np = np.zeros((4, 128), dtype=np.float32)
    for j in range(4):
        p4_np[j, _C * j:_C * (j + 1)] = 1.0
    p4 = jnp.asarray(p4_np)
    r4_np = np.zeros((32, 128), dtype=np.float32)
    for j in range(4):
        for c in range(_C):
            r4_np[c, _C * j + c] = 1.0
    r4 = jnp.asarray(r4_np)
    esrc = edge_index[0].astype(jnp.int32)
    edst = edge_index[1].astype(jnp.int32)
    an32 = atomic_numbers.astype(jnp.int32)

    es, et = _sc_elem_gather(esrc, edst, an32)
    emb_a, emb_b, emb_c4 = _edge_stage(
        edge_distance, es.reshape(e, 1), et.reshape(e, 1), edst.reshape(e, 1),
        wigner_inv.reshape(e, _NUM_COEFF * _NUM_COEFF),
        w1d, s96, t96, w2t, w3rep, e3, vecs, b3rep, p4, r4)
    nchunks = e // _CK
    pad_rows = 16
    dst_pad = jnp.concatenate(
        [edst.reshape(nchunks, 1, _CK),
         jnp.zeros((pad_rows, 1, _CK), jnp.int32)], axis=0)
    nq = ((n_nodes + 3) // 4 + _CK - 1) // _CK * _CK
    dstq_pad = jnp.concatenate(
        [(edst // 4).reshape(nchunks, 1, _CK),
         jnp.zeros((pad_rows, 1, _CK), jnp.int32)], axis=0)
    out_a, out_b = _sc_scatter_main(emb_a, emb_b, dst_pad, n_nodes)
    out_c0, out_c1 = _sc_scatter_rem(emb_c4, dstq_pad, nq)
    out_c = (out_c0 + out_c1)[:n_nodes // 4].reshape(n_nodes, _C)
    out = jnp.concatenate([out_a, out_b, out_c], axis=1)
    return out.reshape(n_nodes, _NUM_COEFF, _C)


# speedup vs baseline: 26.4704x; 1.2435x over previous
import functools

import numpy as np
import jax
import jax.numpy as jnp
from jax import lax
from jax.experimental import pallas as pl
from jax.experimental.pallas import tpu as pltpu
from jax.experimental.pallas import tpu_sc as plsc

_NUM_COEFF = 9
_M0 = 3
_C = 32
_RESCALE = 4.0
_L_IDX = (0, 2, 6)
_HALF = (_NUM_COEFF * _C) // 2


def _sc_elem_gather(esrc, edst, an):
    e = esrc.shape[0]
    n = an.shape[0]
    per = e // 16
    mesh = plsc.VectorSubcoreMesh(core_axis_name="c", subcore_axis_name="s")

    @functools.partial(
        pl.kernel,
        out_type=[jax.ShapeDtypeStruct((e,), jnp.int32),
                  jax.ShapeDtypeStruct((e,), jnp.int32)],
        mesh=mesh,
        scratch_types=[pltpu.VMEM((per,), jnp.int32),
                       pltpu.VMEM((per,), jnp.int32),
                       pltpu.SemaphoreType.DMA],
    )
    def k(esrc_h, edst_h, an_h, es_h, et_h, idx_v, out_v, sem):
        cid = lax.axis_index("c")
        sid = lax.axis_index("s")
        base = sid * per

        @pl.when(cid == 0)
        def _():
            pltpu.sync_copy(esrc_h.at[pl.ds(base, per)], idx_v)

        @pl.when(cid == 1)
        def _():
            pltpu.sync_copy(edst_h.at[pl.ds(base, per)], idx_v)

        pltpu.async_copy(an_h.at[idx_v], out_v, sem).wait()

        @pl.when(cid == 0)
        def _():
            pltpu.sync_copy(out_v, es_h.at[pl.ds(base, per)])

        @pl.when(cid == 1)
        def _():
            pltpu.sync_copy(out_v, et_h.at[pl.ds(base, per)])

    return k(esrc, edst, an)


def _ln(h, g, b):
    mu = jnp.mean(h, axis=-1, keepdims=True)
    var = jnp.mean((h - mu) ** 2, axis=-1, keepdims=True)
    return (h - mu) * jax.lax.rsqrt(var + 1e-5) * g + b


def _edge_body(dist_ref, es_ref, et_ref, ed_ref, wig_ref,
               w1d_ref, s_ref, t_ref, w2_ref, w3rep_ref, e3_ref,
               vec_ref, b3rep_ref, p4_ref, r4_ref,
               lo_ref, hi_ref, rem_ref):
    blk = dist_ref.shape[0]
    b1 = vec_ref[0, :]
    g1 = vec_ref[1, :]
    be1 = vec_ref[2, :]
    b2 = vec_ref[3, :]
    g2 = vec_ref[4, :]
    be2 = vec_ref[5, :]
    ncls = s_ref.shape[0]
    tn = (((0,), (0,)), ((), ()))
    cls_iota = jax.lax.broadcasted_iota(jnp.int32, (ncls, blk), 0)
    oh_s = (es_ref[...] == cls_iota).astype(jnp.float32)
    oh_t = (et_ref[...] == cls_iota).astype(jnp.float32)
    h = (jnp.dot(dist_ref[...], w1d_ref[...], preferred_element_type=jnp.float32)
         + jax.lax.dot_general(oh_s, s_ref[...], tn,
                               preferred_element_type=jnp.float32)
         + jax.lax.dot_general(oh_t, t_ref[...], tn,
                               preferred_element_type=jnp.float32)
         + b1)
    h = jax.nn.silu(_ln(h, g1, be1))
    h = jnp.dot(h, w2_ref[...], preferred_element_type=jnp.float32) + b2
    h = jax.nn.silu(_ln(h, g2, be2))
    wig = wig_ref[...]
    emb = None
    for k in range(_M0):
        hk = jnp.dot(h, w3rep_ref[k], preferred_element_type=jnp.float32) \
            + b3rep_ref[k, :]
        wk = jnp.dot(wig, e3_ref[k], preferred_element_type=jnp.float32)
        term = wk * hk
        emb = term if emb is None else emb + term
    lo_ref[...] = emb[:, 0:128]
    hi_ref[...] = emb[:, 128:256]
    md = jnp.bitwise_and(ed_ref[...], 3)
    oh4 = (md == jax.lax.broadcasted_iota(jnp.int32, (4, blk), 0))
    w4 = jax.lax.dot_general(oh4.astype(jnp.float32), p4_ref[...], tn,
                             preferred_element_type=jnp.float32)
    h4 = jnp.dot(emb[:, 256:288], r4_ref[...],
                 preferred_element_type=jnp.float32)
    rem_ref[...] = w4 * h4


def _edge_stage(dist, es, et, ed, wig_flat, w1d, s96, t96, w2t, w3rep, e3,
                vecs, b3rep, p4, r4):
    e = dist.shape[0]
    blk = 3200
    grid = e // blk
    nc2 = _NUM_COEFF * _NUM_COEFF
    return pl.pallas_call(
        _edge_body,
        grid=(grid,),
        in_specs=[
            pl.BlockSpec((blk, 128), lambda i: (i, 0)),
            pl.BlockSpec((1, blk), lambda i: (0, i)),
            pl.BlockSpec((1, blk), lambda i: (0, i)),
            pl.BlockSpec((1, blk), lambda i: (0, i)),
            pl.BlockSpec((blk, nc2), lambda i: (i, 0)),
            pl.BlockSpec((128, 64), lambda i: (0, 0)),
            pl.BlockSpec((96, 64), lambda i: (0, 0)),
            pl.BlockSpec((96, 64), lambda i: (0, 0)),
            pl.BlockSpec((64, 64), lambda i: (0, 0)),
            pl.BlockSpec((_M0, 64, 288), lambda i: (0, 0, 0)),
            pl.BlockSpec((_M0, nc2, 288), lambda i: (0, 0, 0)),
            pl.BlockSpec((8, 64), lambda i: (0, 0)),
            pl.BlockSpec((_M0, 288), lambda i: (0, 0)),
            pl.BlockSpec((4, 128), lambda i: (0, 0)),
            pl.BlockSpec((32, 128), lambda i: (0, 0)),
        ],
        out_specs=[pl.BlockSpec((blk, 128), lambda i: (i, 0)),
                   pl.BlockSpec((blk, 128), lambda i: (i, 0)),
                   pl.BlockSpec((blk, 128), lambda i: (i, 0))],
        out_shape=[jax.ShapeDtypeStruct((e, 128), jnp.float32),
                   jax.ShapeDtypeStruct((e, 128), jnp.float32),
                   jax.ShapeDtypeStruct((e, 128), jnp.float32)],
    )(dist, es, et, ed, wig_flat, w1d, s96, t96, w2t, w3rep, e3, vecs, b3rep,
      p4, r4)


_CK = 128


def _sc_scatter_main(emb_a, emb_b, dst_pad, n_nodes):
    e = emb_a.shape[0]
    nchunks = e // _CK
    base_per = nchunks // 16
    extra = nchunks - base_per * 16
    per_max = base_per + (1 if extra else 0)
    assert dst_pad.shape[0] >= base_per * 15 + min(15, extra) + per_max
    nfull = n_nodes // _CK
    tail = n_nodes - nfull * _CK
    zeros = jnp.zeros((_CK, 128), jnp.float32)
    mesh = plsc.VectorSubcoreMesh(core_axis_name="c", subcore_axis_name="s")

    @functools.partial(
        pl.kernel,
        out_type=[jax.ShapeDtypeStruct((n_nodes, 128), jnp.float32),
                  jax.ShapeDtypeStruct((n_nodes, 128), jnp.float32)],
        mesh=mesh,
        scratch_types=[pltpu.VMEM_SHARED((n_nodes, 128), jnp.float32),
                       pltpu.VMEM((per_max, 1, _CK), jnp.int32),
                       pltpu.VMEM((2, _CK, 128), jnp.float32),
                       pltpu.VMEM((max(tail, 8), 128), jnp.float32),
                       pltpu.SemaphoreType.DMA,
                       pltpu.SemaphoreType.DMA],
    )
    def k(a_h, b_h, dst_h, zero_h, out_a_h, out_b_h,
          acc, idx_v, rows_v, tail_v, sem0, sem1):
        cid = lax.axis_index("c")
        sid = lax.axis_index("s")
        sems = (sem0, sem1)
        start = base_per * sid + jnp.minimum(sid, extra)
        cnt = base_per + (sid < extra).astype(jnp.int32)
        pltpu.sync_copy(zero_h, rows_v.at[0])
        for z in range((nfull + 15) // 16):
            cidx = sid + 16 * z

            @pl.when(cidx < nfull)
            def _():
                pltpu.sync_copy(rows_v.at[0], acc.at[pl.ds(cidx * _CK, _CK)])

        if tail:
            @pl.when(sid == 0)
            def _():
                pltpu.sync_copy(zero_h.at[pl.ds(0, tail)], tail_v)
                pltpu.sync_copy(tail_v, acc.at[pl.ds(nfull * _CK, tail)])

        pltpu.sync_copy(dst_h.at[pl.ds(start, per_max)], idx_v)
        plsc.subcore_barrier()

        def start_load(j):
            c = start + j
            buf = j % 2

            @pl.when(j < cnt)
            def _():
                @pl.when(cid == 0)
                def _():
                    pltpu.async_copy(a_h.at[pl.ds(c * _CK, _CK)],
                                     rows_v.at[buf], sems[buf])

                @pl.when(cid == 1)
                def _():
                    pltpu.async_copy(b_h.at[pl.ds(c * _CK, _CK)],
                                     rows_v.at[buf], sems[buf])

        start_load(0)
        for j in range(per_max):
            buf = j % 2

            @pl.when(j < cnt)
            def _():
                pltpu.make_async_copy(a_h.at[pl.ds(0, _CK)],
                                      rows_v.at[buf], sems[buf]).wait()

            if j + 1 < per_max:
                start_load(j + 1)

            @pl.when(j < cnt)
            def _():
                pltpu.sync_copy(rows_v.at[buf], acc.at[idx_v.at[j, 0]], add=True)

        plsc.subcore_barrier()
        for z in range((nfull + 15) // 16):
            cidx = sid + 16 * z

            @pl.when(cidx < nfull)
            def _():
                r0 = cidx * _CK
                pltpu.sync_copy(acc.at[pl.ds(r0, _CK)], rows_v.at[0])

                @pl.when(cid == 0)
                def _():
                    pltpu.sync_copy(rows_v.at[0], out_a_h.at[pl.ds(r0, _CK)])

                @pl.when(cid == 1)
                def _():
                    pltpu.sync_copy(rows_v.at[0], out_b_h.at[pl.ds(r0, _CK)])

        if tail:
            @pl.when(sid == 1)
            def _():
                r0 = nfull * _CK
                pltpu.sync_copy(acc.at[pl.ds(r0, tail)], tail_v)

                @pl.when(cid == 0)
                def _():
                    pltpu.sync_copy(tail_v, out_a_h.at[pl.ds(r0, tail)])

                @pl.when(cid == 1)
                def _():
                    pltpu.sync_copy(tail_v, out_b_h.at[pl.ds(r0, tail)])

    return k(emb_a, emb_b, dst_pad, zeros)


def _sc_scatter_rem(emb_c4, dstq_pad, nq):
    e = emb_c4.shape[0]
    nchunks = e // _CK
    half = nchunks // 2
    base_per = half // 16
    extra = half - base_per * 16
    per_max = base_per + (1 if extra else 0)
    nfull = nq // _CK
    assert nq % _CK == 0
    zeros = jnp.zeros((_CK, 128), jnp.float32)
    mesh = plsc.VectorSubcoreMesh(core_axis_name="c", subcore_axis_name="s")

    @functools.partial(
        pl.kernel,
        out_type=[jax.ShapeDtypeStruct((nq, 128), jnp.float32),
                  jax.ShapeDtypeStruct((nq, 128), jnp.float32)],
        mesh=mesh,
        scratch_types=[pltpu.VMEM_SHARED((nq, 128), jnp.float32),
                       pltpu.VMEM((per_max, 1, _CK), jnp.int32),
                       pltpu.VMEM((2, _CK, 128), jnp.float32),
                       pltpu.SemaphoreType.DMA,
                       pltpu.SemaphoreType.DMA],
    )
    def k(c_h, dstq_h, zero_h, out0_h, out1_h, acc, idx_v, rows_v, sem0, sem1):
        cid = lax.axis_index("c")
        sid = lax.axis_index("s")
        sems = (sem0, sem1)
        start = cid * half + base_per * sid + jnp.minimum(sid, extra)
        cnt = base_per + (sid < extra).astype(jnp.int32)
        pltpu.sync_copy(zero_h, rows_v.at[0])
        for z in range((nfull + 15) // 16):
            cidx = sid + 16 * z

            @pl.when(cidx < nfull)
            def _():
                pltpu.sync_copy(rows_v.at[0], acc.at[pl.ds(cidx * _CK, _CK)])

        pltpu.sync_copy(dstq_h.at[pl.ds(start, per_max)], idx_v)
        plsc.subcore_barrier()

        def start_load(j):
            c = start + j
            buf = j % 2

            @pl.when(j < cnt)
            def _():
                pltpu.async_copy(c_h.at[pl.ds(c * _CK, _CK)],
                                 rows_v.at[buf], sems[buf])

        start_load(0)
        for j in range(per_max):
            buf = j % 2

            @pl.when(j < cnt)
            def _():
                pltpu.make_async_copy(c_h.at[pl.ds(0, _CK)],
                                      rows_v.at[buf], sems[buf]).wait()

            if j + 1 < per_max:
                start_load(j + 1)

            @pl.when(j < cnt)
            def _():
                pltpu.sync_copy(rows_v.at[buf], acc.at[idx_v.at[j, 0]], add=True)

        plsc.subcore_barrier()
        for z in range((nfull + 15) // 16):
            cidx = sid + 16 * z

            @pl.when(cidx < nfull)
            def _():
                r0 = cidx * _CK
                pltpu.sync_copy(acc.at[pl.ds(r0, _CK)], rows_v.at[0])

                @pl.when(cid == 0)
                def _():
                    pltpu.sync_copy(rows_v.at[0], out0_h.at[pl.ds(r0, _CK)])

                @pl.when(cid == 1)
                def _():
                    pltpu.sync_copy(rows_v.at[0], out1_h.at[pl.ds(r0, _CK)])

    return k(emb_c4, dstq_pad, zeros)


def kernel(x, atomic_numbers, edge_distance, edge_index, src_table, tgt_table,
           W1, b1, g1, be1, W2, b2, g2, be2, W3, b3, to_m, wigner_inv):
    n_nodes = x.shape[0]
    e = edge_distance.shape[0]
    w1d = W1[:, :128].T
    s96 = jnp.zeros((96, 64), jnp.float32).at[:src_table.shape[0]].set(
        src_table @ W1[:, 128:192].T)
    t96 = jnp.zeros((96, 64), jnp.float32).at[:tgt_table.shape[0]].set(
        tgt_table @ W1[:, 192:256].T)
    w2t = W2.T
    w3t = (W3 / _RESCALE).T
    b3s = (b3 / _RESCALE).reshape(_M0, _C)
    w3rep = jnp.stack([jnp.tile(w3t[:, k * _C:(k + 1) * _C], (1, _NUM_COEFF))
                       for k in range(_M0)], axis=0)
    b3rep = jnp.stack([jnp.tile(b3s[k], (_NUM_COEFF,))
                       for k in range(_M0)], axis=0)
    e3_np = np.zeros((_M0, _NUM_COEFF * _NUM_COEFF, _NUM_COEFF * _C),
                      dtype=np.float32)
    for k, l_idx in enumerate(_L_IDX):
        for i in range(_NUM_COEFF):
            e3_np[k, 9 * i + l_idx, _C * i:_C * (i + 1)] = 1.0
    e3 = jnp.asarray(e3_np)
    vecs = jnp.stack([b1, g1, be1, b2, g2, be2, jnp.zeros_like(b1),
                      jnp.zeros_like(b1)], axis=0)
    p4_np = np.zeros((4, 128), dtype=np.float32)
    for j in range(4):
        p4_np[j, _C * j:_C * (j + 1)] = 1.0
    p4 = jnp.asarray(p4_np)
    r4_np = np.zeros((32, 128), dtype=np.float32)
    for j in range(4):
        for c in range(_C):
            r4_np[c, _C * j + c] = 1.0
    r4 = jnp.asarray(r4_np)
    esrc = edge_index[0].astype(jnp.int32)
    edst = edge_index[1].astype(jnp.int32)
    an32 = atomic_numbers.astype(jnp.int32)

    es, et = _sc_elem_gather(esrc, edst, an32)
    emb_a, emb_b, emb_c4 = _edge_stage(
        edge_distance, es.reshape(1, e), et.reshape(1, e), edst.reshape(1, e),
        wigner_inv.reshape(e, _NUM_COEFF * _NUM_COEFF),
        w1d, s96, t96, w2t, w3rep, e3, vecs, b3rep, p4, r4)
    nchunks = e // _CK
    pad_rows = 16
    dst_pad = jnp.concatenate(
        [edst.reshape(nchunks, 1, _CK),
         jnp.zeros((pad_rows, 1, _CK), jnp.int32)], axis=0)
    nq = ((n_nodes + 3) // 4 + _CK - 1) // _CK * _CK
    dstq_pad = jnp.concatenate(
        [(edst // 4).reshape(nchunks, 1, _CK),
         jnp.zeros((pad_rows, 1, _CK), jnp.int32)], axis=0)
    out_a, out_b = _sc_scatter_main(emb_a, emb_b, dst_pad, n_nodes)
    out_c0, out_c1 = _sc_scatter_rem(emb_c4, dstq_pad, nq)
    out_c = (out_c0 + out_c1)[:n_nodes // 4].reshape(n_nodes, _C)
    out = jnp.concatenate([out_a, out_b, out_c], axis=1)
    return out.reshape(n_nodes, _NUM_COEFF, _C)
